# Initial kernel scaffold; baseline (speedup 1.0000x reference)
import jax
import jax.numpy as jnp
from jax.experimental import pallas as pl


def kernel(x, edge_index):
    raise NotImplementedError("write your pallas kernel here")


# speedup vs baseline: 1.4441x; 1.4441x over previous
import functools

import jax
import jax.numpy as jnp
from jax import lax
from jax.experimental import pallas as pl
from jax.experimental.pallas import tpu as pltpu
from jax.experimental.pallas import tpu_sc as plsc

N_NODES_PAD = 10240
EDGES = 320000
NC, NS = 2, 16
CHUNK = 80
ROWS_PER_SUBCORE = EDGES // (NS * CHUNK)
SLICE = N_NODES_PAD // NS

_mesh = plsc.VectorSubcoreMesh(
    core_axis_name="c", subcore_axis_name="s", num_cores=NC, num_subcores=NS
)


@functools.partial(
    pl.kernel,
    out_type=jax.ShapeDtypeStruct((NC, N_NODES_PAD), jnp.int32),
    mesh=_mesh,
    scratch_types=[
        pltpu.VMEM((ROWS_PER_SUBCORE, CHUNK), jnp.int32),
        pltpu.VMEM((CHUNK,), jnp.int32),
        pltpu.VMEM((SLICE,), jnp.int32),
        pltpu.VMEM_SHARED((N_NODES_PAD,), jnp.int32),
    ],
    compiler_params=pltpu.CompilerParams(use_tc_tiling_on_sc=False),
)
def _degree_sc(edge_hbm, deg_hbm, idx_v, ones_v, zero_v, hist_s):
    c = lax.axis_index("c")
    s = lax.axis_index("s")

    for k in range(CHUNK // 16):
        ones_v[pl.ds(k * 16, 16)] = jnp.full((16,), 1, jnp.int32)
    for k in range(SLICE // 16):
        zero_v[pl.ds(k * 16, 16)] = jnp.zeros((16,), jnp.int32)

    pltpu.sync_copy(edge_hbm.at[c, pl.ds(s * ROWS_PER_SUBCORE, ROWS_PER_SUBCORE)],
                    idx_v)
    pltpu.sync_copy(zero_v, hist_s.at[pl.ds(s * SLICE, SLICE)])
    plsc.subcore_barrier()

    def body(j, carry):
        pltpu.sync_copy(ones_v, hist_s.at[idx_v.at[j]], add=True)
        return carry

    lax.fori_loop(0, ROWS_PER_SUBCORE, body, 0)
    plsc.subcore_barrier()

    pltpu.sync_copy(hist_s.at[pl.ds(s * SLICE, SLICE)],
                    deg_hbm.at[c, pl.ds(s * SLICE, SLICE)])


def kernel(x, edge_index):
    ei = edge_index.astype(jnp.int32).reshape(NC, EDGES // CHUNK, CHUNK)
    deg = _degree_sc(ei)
    out_dtype = jax.dtypes.canonicalize_dtype(jnp.int64)
    out_degree = deg[0, :10000].astype(out_dtype)
    in_degree = deg[1, :10000].astype(out_dtype)
    return x, in_degree, out_degree


# speedup vs baseline: 1.8859x; 1.3060x over previous
import functools

import jax
import jax.numpy as jnp
from jax import lax
from jax.experimental import pallas as pl
from jax.experimental.pallas import tpu as pltpu
from jax.experimental.pallas import tpu_sc as plsc

N_NODES_PAD = 10240
EDGES = 320000
NC, NS = 2, 16
PER_SUBCORE = EDGES // NS
SLICE = N_NODES_PAD // NS

_mesh = plsc.VectorSubcoreMesh(
    core_axis_name="c", subcore_axis_name="s", num_cores=NC, num_subcores=NS
)


@functools.partial(
    pl.kernel,
    out_type=jax.ShapeDtypeStruct((NC, N_NODES_PAD), jnp.int32),
    mesh=_mesh,
    scratch_types=[
        pltpu.VMEM((PER_SUBCORE,), jnp.int32),
        pltpu.VMEM((PER_SUBCORE,), jnp.int32),
        pltpu.VMEM((SLICE,), jnp.int32),
        pltpu.VMEM_SHARED((N_NODES_PAD,), jnp.int32),
        pltpu.SemaphoreType.DMA,
    ],
    compiler_params=pltpu.CompilerParams(use_tc_tiling_on_sc=False),
)
def _degree_sc(edge_hbm, deg_hbm, idx_v, ones_v, zero_v, hist_s, sem):
    c = lax.axis_index("c")
    s = lax.axis_index("s")

    idx_cp = pltpu.async_copy(
        edge_hbm.at[c, pl.ds(s * PER_SUBCORE, PER_SUBCORE)], idx_v, sem
    )

    def fill_ones(i, carry):
        ones_v[pl.ds(pl.multiple_of(i * 16, 16), 16)] = jnp.full((16,), 1, jnp.int32)
        return carry

    lax.fori_loop(0, PER_SUBCORE // 16, fill_ones, 0)
    for k in range(SLICE // 16):
        zero_v[pl.ds(k * 16, 16)] = jnp.zeros((16,), jnp.int32)

    pltpu.sync_copy(zero_v, hist_s.at[pl.ds(s * SLICE, SLICE)])
    idx_cp.wait()
    plsc.subcore_barrier()

    pltpu.sync_copy(ones_v, hist_s.at[idx_v], add=True)
    plsc.subcore_barrier()

    pltpu.sync_copy(hist_s.at[pl.ds(s * SLICE, SLICE)],
                    deg_hbm.at[c, pl.ds(s * SLICE, SLICE)])


def kernel(x, edge_index):
    ei = edge_index.astype(jnp.int32)
    deg = _degree_sc(ei)
    out_dtype = jax.dtypes.canonicalize_dtype(jnp.int64)
    out_degree = deg[0, :10000].astype(out_dtype)
    in_degree = deg[1, :10000].astype(out_dtype)
    return x, in_degree, out_degree
